# Initial kernel scaffold; baseline (speedup 1.0000x reference)
import jax
import jax.numpy as jnp
from jax.experimental import pallas as pl


def kernel(x, edge_index, batch, W1, b1, W2, b2, W3, b3):
    raise NotImplementedError("write your pallas kernel here")


# speedup vs baseline: 6.9293x; 6.9293x over previous
import functools

import jax
import jax.numpy as jnp
from jax import lax
from jax.experimental import pallas as pl
from jax.experimental.pallas import tpu as pltpu
from jax.experimental.pallas import tpu_sc as plsc

N = 10000
E = 320000
D = 128
G = 16

NC = 2
NS = 16
NW = NC * NS
EB = 128
N_PAD = 10240
EDGES_PER_TILE = 10240
E_PAD = NW * EDGES_PER_TILE
BLOCKS_PER_TILE = EDGES_PER_TILE // EB
ROWS_PER_TILE = N_PAD // NS
def _sc_mesh():
    return plsc.VectorSubcoreMesh(core_axis_name="c", subcore_axis_name="s")


def _deg_sc(dst_p, ones_blk, zeros_blk):
    @functools.partial(
        pl.kernel,
        mesh=_sc_mesh(),
        out_type=jax.ShapeDtypeStruct((NC, N_PAD, D), jnp.float32),
        scratch_types=[
            pltpu.VMEM((EB,), jnp.int32),
            pltpu.VMEM((EB, D), jnp.float32),
            pltpu.VMEM_SHARED((N_PAD, D), jnp.float32),
        ],
    )
    def k(dst_hbm, ones_hbm, z_hbm, out_hbm, didx, ones_v, acc):
        cid = lax.axis_index("c")
        sid = lax.axis_index("s")
        wid = cid * NS + sid
        pltpu.sync_copy(ones_hbm, ones_v)
        pltpu.sync_copy(z_hbm, acc.at[pl.ds(sid * ROWS_PER_TILE, ROWS_PER_TILE)])
        plsc.subcore_barrier()
        base = wid * EDGES_PER_TILE

        @pl.loop(0, BLOCKS_PER_TILE)
        def _(b):
            pltpu.sync_copy(dst_hbm.at[pl.ds(base + b * EB, EB)], didx)
            pltpu.sync_copy(ones_v, acc.at[didx], add=True)

        plsc.subcore_barrier()
        pltpu.sync_copy(
            acc.at[pl.ds(sid * ROWS_PER_TILE, ROWS_PER_TILE)],
            out_hbm.at[cid, pl.ds(sid * ROWS_PER_TILE, ROWS_PER_TILE)],
        )

    return k(dst_p, ones_blk, zeros_blk)


def _spmm_sc(g_table, src_p, dst_p, zeros_blk):
    @functools.partial(
        pl.kernel,
        mesh=_sc_mesh(),
        out_type=jax.ShapeDtypeStruct((NC, N_PAD, D), jnp.float32),
        scratch_types=[
            pltpu.VMEM((EB,), jnp.int32),
            pltpu.VMEM((EB,), jnp.int32),
            pltpu.VMEM((EB, D), jnp.float32),
            pltpu.VMEM_SHARED((N_PAD, D), jnp.float32),
            pltpu.SemaphoreType.DMA,
        ],
    )
    def k(g_hbm, src_hbm, dst_hbm, z_hbm, out_hbm, sidx, didx, rows, acc, sem):
        cid = lax.axis_index("c")
        sid = lax.axis_index("s")
        wid = cid * NS + sid
        pltpu.sync_copy(z_hbm, acc.at[pl.ds(sid * ROWS_PER_TILE, ROWS_PER_TILE)])
        plsc.subcore_barrier()
        base = wid * EDGES_PER_TILE

        @pl.loop(0, BLOCKS_PER_TILE)
        def _(b):
            off = base + b * EB
            pltpu.sync_copy(src_hbm.at[pl.ds(off, EB)], sidx)
            pltpu.sync_copy(dst_hbm.at[pl.ds(off, EB)], didx)
            pltpu.async_copy(g_hbm.at[sidx], rows, sem).wait()
            pltpu.sync_copy(rows, acc.at[didx], add=True)

        plsc.subcore_barrier()
        pltpu.sync_copy(
            acc.at[pl.ds(sid * ROWS_PER_TILE, ROWS_PER_TILE)],
            out_hbm.at[cid, pl.ds(sid * ROWS_PER_TILE, ROWS_PER_TILE)],
        )

    return k(g_table, src_p, dst_p, zeros_blk)


def _tc_pre(x, W, degp):
    def body(x_ref, w_ref, deg_ref, h_ref, g_ref, dis_ref, inv_ref):
        deg = deg_ref[0][:N, 0:1] + deg_ref[1][:N, 0:1] + 1.0
        dis = lax.rsqrt(deg)
        h = jnp.dot(x_ref[...], w_ref[...], preferred_element_type=jnp.float32)
        h_ref[...] = h
        g_ref[...] = h * dis
        dis_ref[...] = dis
        inv_ref[...] = 1.0 / deg

    return pl.pallas_call(
        body,
        out_shape=(
            jax.ShapeDtypeStruct((N, D), jnp.float32),
            jax.ShapeDtypeStruct((N, D), jnp.float32),
            jax.ShapeDtypeStruct((N, 1), jnp.float32),
            jax.ShapeDtypeStruct((N, 1), jnp.float32),
        ),
    )(x, W, degp)


def _tc_mid(p, h_prev, dis, inv, b_prev, W_next):
    def body(p_ref, h_ref, dis_ref, inv_ref, b_ref, w_ref, ho_ref, go_ref):
        acc = p_ref[0][:N, :] + p_ref[1][:N, :]
        out = dis_ref[...] * acc + inv_ref[...] * h_ref[...] + b_ref[...][None, :]
        act = jnp.maximum(out, 0.0)
        h = jnp.dot(act, w_ref[...], preferred_element_type=jnp.float32)
        ho_ref[...] = h
        go_ref[...] = h * dis_ref[...]

    return pl.pallas_call(
        body,
        out_shape=(
            jax.ShapeDtypeStruct((N, D), jnp.float32),
            jax.ShapeDtypeStruct((N, D), jnp.float32),
        ),
    )(p, h_prev, dis, inv, b_prev, W_next)


def _tc_post(p, h_prev, dis, inv, b, batch):
    def body(p_ref, h_ref, dis_ref, inv_ref, b_ref, batch_ref, o_ref):
        acc = p_ref[0][:N, :] + p_ref[1][:N, :]
        out = dis_ref[...] * acc + inv_ref[...] * h_ref[...] + b_ref[...][None, :]
        bat = batch_ref[...]
        neg = jnp.float32(-jnp.inf)
        rows = []
        for gi in range(G):
            m = bat == gi
            rows.append(jnp.max(jnp.where(m, out, neg), axis=0, keepdims=True))
        o_ref[...] = jnp.concatenate(rows, axis=0)

    return pl.pallas_call(
        body,
        out_shape=jax.ShapeDtypeStruct((G, D), jnp.float32),
    )(p, h_prev, dis, inv, b, batch.reshape(N, 1))


def kernel(x, edge_index, batch, W1, b1, W2, b2, W3, b3):
    src = edge_index[0]
    dst = edge_index[1]
    pad_e = E_PAD - E
    src_p = jnp.concatenate([src, jnp.zeros((pad_e,), jnp.int32)])
    dst_p = jnp.concatenate([dst, jnp.full((pad_e,), N_PAD - 1, jnp.int32)])
    zeros_blk = jnp.zeros((ROWS_PER_TILE, D), jnp.float32)
    ones_blk = jnp.ones((EB, D), jnp.float32)

    degp = _deg_sc(dst_p, ones_blk, zeros_blk)
    h1, g1, dis, inv = _tc_pre(x, W1, degp)
    p1 = _spmm_sc(g1, src_p, dst_p, zeros_blk)
    h2, g2 = _tc_mid(p1, h1, dis, inv, b1, W2)
    p2 = _spmm_sc(g2, src_p, dst_p, zeros_blk)
    h3, g3 = _tc_mid(p2, h2, dis, inv, b2, W3)
    p3 = _spmm_sc(g3, src_p, dst_p, zeros_blk)
    return _tc_post(p3, h3, dis, inv, b3, batch)


# speedup vs baseline: 9.0188x; 1.3015x over previous
import functools

import jax
import jax.numpy as jnp
from jax import lax
from jax.experimental import pallas as pl
from jax.experimental.pallas import tpu as pltpu
from jax.experimental.pallas import tpu_sc as plsc

N = 10000
E = 320000
D = 128
G = 16

NC = 2
NS = 16
NW = NC * NS
EB = 128
N_PAD = 10240
EDGES_PER_TILE = 10240
E_PAD = NW * EDGES_PER_TILE
BLOCKS_PER_TILE = EDGES_PER_TILE // EB
CHUNK = 20
NCHUNK = BLOCKS_PER_TILE // CHUNK
ROWS_PER_TILE = N_PAD // NS
def _sc_mesh():
    return plsc.VectorSubcoreMesh(core_axis_name="c", subcore_axis_name="s")


def _deg_sc(dst_p, ones_blk, zeros_blk):
    @functools.partial(
        pl.kernel,
        mesh=_sc_mesh(),
        out_type=jax.ShapeDtypeStruct((NC, N_PAD, D), jnp.float32),
        scratch_types=[
            pltpu.VMEM((EB,), jnp.int32),
            pltpu.VMEM((EB, D), jnp.float32),
            pltpu.VMEM_SHARED((N_PAD, D), jnp.float32),
        ],
    )
    def k(dst_hbm, ones_hbm, z_hbm, out_hbm, didx, ones_v, acc):
        cid = lax.axis_index("c")
        sid = lax.axis_index("s")
        wid = cid * NS + sid
        pltpu.sync_copy(ones_hbm, ones_v)
        pltpu.sync_copy(z_hbm, acc.at[pl.ds(sid * ROWS_PER_TILE, ROWS_PER_TILE)])
        plsc.subcore_barrier()
        base = wid * EDGES_PER_TILE

        @pl.loop(0, BLOCKS_PER_TILE)
        def _(b):
            pltpu.sync_copy(dst_hbm.at[pl.ds(base + b * EB, EB)], didx)
            pltpu.sync_copy(ones_v, acc.at[didx], add=True)

        plsc.subcore_barrier()
        pltpu.sync_copy(
            acc.at[pl.ds(sid * ROWS_PER_TILE, ROWS_PER_TILE)],
            out_hbm.at[cid, pl.ds(sid * ROWS_PER_TILE, ROWS_PER_TILE)],
        )

    return k(dst_p, ones_blk, zeros_blk)


def _spmm_sc(g_table, src_p, dst_p, zeros_blk):
    @functools.partial(
        pl.kernel,
        mesh=_sc_mesh(),
        out_type=jax.ShapeDtypeStruct((NC, N_PAD, D), jnp.float32),
        scratch_types=[
            pltpu.VMEM((EB,), jnp.int32),
            pltpu.VMEM((EB,), jnp.int32),
            pltpu.VMEM((EB,), jnp.int32),
            pltpu.VMEM((EB,), jnp.int32),
            pltpu.VMEM((EB,), jnp.int32),
            pltpu.VMEM((EB,), jnp.int32),
            pltpu.VMEM((EB,), jnp.int32),
            pltpu.VMEM((EB,), jnp.int32),
            pltpu.VMEM((EB, D), jnp.float32),
            pltpu.VMEM((EB, D), jnp.float32),
            pltpu.VMEM_SHARED((N_PAD, D), jnp.float32),
            pltpu.SemaphoreType.DMA,
            pltpu.SemaphoreType.DMA,
            pltpu.SemaphoreType.DMA,
            pltpu.SemaphoreType.DMA,
            pltpu.SemaphoreType.DMA,
            pltpu.SemaphoreType.DMA,
        ],
    )
    def k(g_hbm, src_hbm, dst_hbm, z_hbm, out_hbm,
          si0, si1, si2, si3, di0, di1, di2, di3, rows0, rows1, acc,
          is0, is1, is2, is3, gs0, gs1):
        sidx = (si0, si1, si2, si3)
        didx = (di0, di1, di2, di3)
        isem = (is0, is1, is2, is3)
        rows = (rows0, rows1)
        gsem = (gs0, gs1)
        cid = lax.axis_index("c")
        sid = lax.axis_index("s")
        wid = cid * NS + sid
        off = wid * EDGES_PER_TILE
        nb = BLOCKS_PER_TILE

        def load_idx(t, s):
            pltpu.async_copy(src_hbm.at[pl.ds(off + t * EB, EB)], sidx[s], isem[s])
            pltpu.async_copy(dst_hbm.at[pl.ds(off + t * EB, EB)], didx[s], isem[s])

        def wait_idx(s):
            pltpu.make_async_copy(src_hbm.at[pl.ds(0, EB)], sidx[s], isem[s]).wait()
            pltpu.make_async_copy(src_hbm.at[pl.ds(0, EB)], didx[s], isem[s]).wait()

        def start_gather(s, r):
            pltpu.async_copy(g_hbm.at[sidx[s]], rows[r], gsem[r])

        def wait_gather_scatter(s, r):
            pltpu.make_async_copy(g_hbm.at[sidx[s]], rows[r], gsem[r]).wait()
            pltpu.sync_copy(rows[r], acc.at[didx[s]], add=True)

        pltpu.sync_copy(z_hbm, acc.at[pl.ds(sid * ROWS_PER_TILE, ROWS_PER_TILE)])
        load_idx(0, 0)
        load_idx(1, 1)
        wait_idx(0)
        start_gather(0, 0)
        plsc.subcore_barrier()

        @pl.loop(0, nb - 4, step=4)
        def _(b):
            for j in range(4):
                t = b + j
                load_idx(t + 2, (j + 2) % 4)
                wait_idx((j + 1) % 4)
                start_gather((j + 1) % 4, (j + 1) % 2)
                wait_gather_scatter(j % 4, j % 2)

        load_idx(nb - 2, 2)
        wait_idx(1)
        start_gather(1, 1)
        wait_gather_scatter(0, 0)

        load_idx(nb - 1, 3)
        wait_idx(2)
        start_gather(2, 0)
        wait_gather_scatter(1, 1)

        wait_idx(3)
        start_gather(3, 1)
        wait_gather_scatter(2, 0)

        wait_gather_scatter(3, 1)

        plsc.subcore_barrier()
        pltpu.sync_copy(
            acc.at[pl.ds(sid * ROWS_PER_TILE, ROWS_PER_TILE)],
            out_hbm.at[cid, pl.ds(sid * ROWS_PER_TILE, ROWS_PER_TILE)],
        )

    return k(g_table, src_p, dst_p, zeros_blk)


def _tc_pre(x, W, degp):
    def body(x_ref, w_ref, deg_ref, h_ref, g_ref, dis_ref, inv_ref):
        deg = deg_ref[0][:N, 0:1] + deg_ref[1][:N, 0:1] + 1.0
        dis = lax.rsqrt(deg)
        h = jnp.dot(x_ref[...], w_ref[...], preferred_element_type=jnp.float32)
        h_ref[...] = h
        g_ref[...] = h * dis
        dis_ref[...] = dis
        inv_ref[...] = 1.0 / deg

    return pl.pallas_call(
        body,
        out_shape=(
            jax.ShapeDtypeStruct((N, D), jnp.float32),
            jax.ShapeDtypeStruct((N, D), jnp.float32),
            jax.ShapeDtypeStruct((N, 1), jnp.float32),
            jax.ShapeDtypeStruct((N, 1), jnp.float32),
        ),
    )(x, W, degp)


def _tc_mid(p, h_prev, dis, inv, b_prev, W_next):
    def body(p_ref, h_ref, dis_ref, inv_ref, b_ref, w_ref, ho_ref, go_ref):
        acc = p_ref[0][:N, :] + p_ref[1][:N, :]
        out = dis_ref[...] * acc + inv_ref[...] * h_ref[...] + b_ref[...][None, :]
        act = jnp.maximum(out, 0.0)
        h = jnp.dot(act, w_ref[...], preferred_element_type=jnp.float32)
        ho_ref[...] = h
        go_ref[...] = h * dis_ref[...]

    return pl.pallas_call(
        body,
        out_shape=(
            jax.ShapeDtypeStruct((N, D), jnp.float32),
            jax.ShapeDtypeStruct((N, D), jnp.float32),
        ),
    )(p, h_prev, dis, inv, b_prev, W_next)


def _tc_post(p, h_prev, dis, inv, b, batch):
    def body(p_ref, h_ref, dis_ref, inv_ref, b_ref, batch_ref, o_ref):
        acc = p_ref[0][:N, :] + p_ref[1][:N, :]
        out = dis_ref[...] * acc + inv_ref[...] * h_ref[...] + b_ref[...][None, :]
        bat = batch_ref[...]
        neg = jnp.float32(-jnp.inf)
        rows = []
        for gi in range(G):
            m = bat == gi
            rows.append(jnp.max(jnp.where(m, out, neg), axis=0, keepdims=True))
        o_ref[...] = jnp.concatenate(rows, axis=0)

    return pl.pallas_call(
        body,
        out_shape=jax.ShapeDtypeStruct((G, D), jnp.float32),
    )(p, h_prev, dis, inv, b, batch.reshape(N, 1))


def kernel(x, edge_index, batch, W1, b1, W2, b2, W3, b3):
    src = edge_index[0]
    dst = edge_index[1]
    pad_e = E_PAD - E
    src_p = jnp.concatenate([src, jnp.zeros((pad_e,), jnp.int32)])
    dst_p = jnp.concatenate([dst, jnp.full((pad_e,), N_PAD - 1, jnp.int32)])
    zeros_blk = jnp.zeros((ROWS_PER_TILE, D), jnp.float32)
    ones_blk = jnp.ones((EB, D), jnp.float32)

    degp = _deg_sc(dst_p, ones_blk, zeros_blk)
    h1, g1, dis, inv = _tc_pre(x, W1, degp)
    p1 = _spmm_sc(g1, src_p, dst_p, zeros_blk)
    h2, g2 = _tc_mid(p1, h1, dis, inv, b1, W2)
    p2 = _spmm_sc(g2, src_p, dst_p, zeros_blk)
    h3, g3 = _tc_mid(p2, h2, dis, inv, b2, W3)
    p3 = _spmm_sc(g3, src_p, dst_p, zeros_blk)
    return _tc_post(p3, h3, dis, inv, b3, batch)


# speedup vs baseline: 9.1106x; 1.0102x over previous
import functools

import jax
import jax.numpy as jnp
from jax import lax
from jax.experimental import pallas as pl
from jax.experimental.pallas import tpu as pltpu
from jax.experimental.pallas import tpu_sc as plsc

N = 10000
E = 320000
D = 128
G = 16

NC = 2
NS = 16
NW = NC * NS
EB = 128
N_PAD = 10240
EDGES_PER_TILE = 10240
E_PAD = NW * EDGES_PER_TILE
BLOCKS_PER_TILE = EDGES_PER_TILE // EB
ROWS_PER_TILE = N_PAD // NS
FAST_CORE = 0
B_FAST = 120
B_SLOW = 2 * BLOCKS_PER_TILE - B_FAST
def _sc_mesh():
    return plsc.VectorSubcoreMesh(core_axis_name="c", subcore_axis_name="s")


def _deg_sc(dst_p, ones_blk, zeros_blk):
    @functools.partial(
        pl.kernel,
        mesh=_sc_mesh(),
        out_type=jax.ShapeDtypeStruct((NC, N_PAD, D), jnp.float32),
        scratch_types=[
            pltpu.VMEM((EB,), jnp.int32),
            pltpu.VMEM((EB, D), jnp.float32),
            pltpu.VMEM_SHARED((N_PAD, D), jnp.float32),
        ],
    )
    def k(dst_hbm, ones_hbm, z_hbm, out_hbm, didx, ones_v, acc):
        cid = lax.axis_index("c")
        sid = lax.axis_index("s")
        wid = cid * NS + sid
        pltpu.sync_copy(ones_hbm, ones_v)
        pltpu.sync_copy(z_hbm, acc.at[pl.ds(sid * ROWS_PER_TILE, ROWS_PER_TILE)])
        plsc.subcore_barrier()
        base = wid * EDGES_PER_TILE

        @pl.loop(0, BLOCKS_PER_TILE)
        def _(b):
            pltpu.sync_copy(dst_hbm.at[pl.ds(base + b * EB, EB)], didx)
            pltpu.sync_copy(ones_v, acc.at[didx], add=True)

        plsc.subcore_barrier()
        pltpu.sync_copy(
            acc.at[pl.ds(sid * ROWS_PER_TILE, ROWS_PER_TILE)],
            out_hbm.at[cid, pl.ds(sid * ROWS_PER_TILE, ROWS_PER_TILE)],
        )

    return k(dst_p, ones_blk, zeros_blk)


def _spmm_sc(g_table, src_p, dst_p, zeros_blk):
    @functools.partial(
        pl.kernel,
        mesh=_sc_mesh(),
        out_type=jax.ShapeDtypeStruct((NC, N_PAD, D), jnp.float32),
        scratch_types=[
            pltpu.VMEM((EB,), jnp.int32),
            pltpu.VMEM((EB,), jnp.int32),
            pltpu.VMEM((EB,), jnp.int32),
            pltpu.VMEM((EB,), jnp.int32),
            pltpu.VMEM((EB,), jnp.int32),
            pltpu.VMEM((EB,), jnp.int32),
            pltpu.VMEM((EB,), jnp.int32),
            pltpu.VMEM((EB,), jnp.int32),
            pltpu.VMEM((EB, D), jnp.float32),
            pltpu.VMEM((EB, D), jnp.float32),
            pltpu.VMEM_SHARED((N_PAD, D), jnp.float32),
            pltpu.SemaphoreType.DMA,
            pltpu.SemaphoreType.DMA,
            pltpu.SemaphoreType.DMA,
            pltpu.SemaphoreType.DMA,
            pltpu.SemaphoreType.DMA,
            pltpu.SemaphoreType.DMA,
        ],
    )
    def k(g_hbm, src_hbm, dst_hbm, z_hbm, out_hbm,
          si0, si1, si2, si3, di0, di1, di2, di3, rows0, rows1, acc,
          is0, is1, is2, is3, gs0, gs1):
        sidx = (si0, si1, si2, si3)
        didx = (di0, di1, di2, di3)
        isem = (is0, is1, is2, is3)
        rows = (rows0, rows1)
        gsem = (gs0, gs1)
        cid = lax.axis_index("c")
        sid = lax.axis_index("s")

        def pipeline(first_block, nb):
            off = first_block * EB

            def load_idx(t, s):
                pltpu.async_copy(src_hbm.at[pl.ds(off + t * EB, EB)], sidx[s], isem[s])
                pltpu.async_copy(dst_hbm.at[pl.ds(off + t * EB, EB)], didx[s], isem[s])

            def wait_idx(s):
                pltpu.make_async_copy(src_hbm.at[pl.ds(0, EB)], sidx[s], isem[s]).wait()
                pltpu.make_async_copy(src_hbm.at[pl.ds(0, EB)], didx[s], isem[s]).wait()

            def start_gather(s, r):
                pltpu.async_copy(g_hbm.at[sidx[s]], rows[r], gsem[r])

            def wait_gather_scatter(s, r):
                pltpu.make_async_copy(g_hbm.at[sidx[s]], rows[r], gsem[r]).wait()
                pltpu.sync_copy(rows[r], acc.at[didx[s]], add=True)

            load_idx(0, 0)
            load_idx(1, 1)
            wait_idx(0)
            start_gather(0, 0)

            @pl.loop(0, nb - 4, step=4)
            def _(b):
                for j in range(4):
                    t = b + j
                    load_idx(t + 2, (j + 2) % 4)
                    wait_idx((j + 1) % 4)
                    start_gather((j + 1) % 4, (j + 1) % 2)
                    wait_gather_scatter(j % 4, j % 2)

            load_idx(nb - 2, 2)
            wait_idx(1)
            start_gather(1, 1)
            wait_gather_scatter(0, 0)

            load_idx(nb - 1, 3)
            wait_idx(2)
            start_gather(2, 0)
            wait_gather_scatter(1, 1)

            wait_idx(3)
            start_gather(3, 1)
            wait_gather_scatter(2, 0)

            wait_gather_scatter(3, 1)

        pltpu.sync_copy(z_hbm, acc.at[pl.ds(sid * ROWS_PER_TILE, ROWS_PER_TILE)])
        plsc.subcore_barrier()

        @pl.when(cid == FAST_CORE)
        def _():
            pipeline(sid * B_FAST, B_FAST)

        @pl.when(cid != FAST_CORE)
        def _():
            pipeline(NS * B_FAST + sid * B_SLOW, B_SLOW)

        plsc.subcore_barrier()
        pltpu.sync_copy(
            acc.at[pl.ds(sid * ROWS_PER_TILE, ROWS_PER_TILE)],
            out_hbm.at[cid, pl.ds(sid * ROWS_PER_TILE, ROWS_PER_TILE)],
        )

    return k(g_table, src_p, dst_p, zeros_blk)


def _tc_pre(x, W, degp):
    def body(x_ref, w_ref, deg_ref, h_ref, g_ref, dis_ref, inv_ref):
        deg = deg_ref[0][:N, 0:1] + deg_ref[1][:N, 0:1] + 1.0
        dis = lax.rsqrt(deg)
        h = jnp.dot(x_ref[...], w_ref[...], preferred_element_type=jnp.float32)
        h_ref[...] = h
        g_ref[...] = h * dis
        dis_ref[...] = dis
        inv_ref[...] = 1.0 / deg

    return pl.pallas_call(
        body,
        out_shape=(
            jax.ShapeDtypeStruct((N, D), jnp.float32),
            jax.ShapeDtypeStruct((N, D), jnp.float32),
            jax.ShapeDtypeStruct((N, 1), jnp.float32),
            jax.ShapeDtypeStruct((N, 1), jnp.float32),
        ),
    )(x, W, degp)


def _tc_mid(p, h_prev, dis, inv, b_prev, W_next):
    def body(p_ref, h_ref, dis_ref, inv_ref, b_ref, w_ref, ho_ref, go_ref):
        acc = p_ref[0][:N, :] + p_ref[1][:N, :]
        out = dis_ref[...] * acc + inv_ref[...] * h_ref[...] + b_ref[...][None, :]
        act = jnp.maximum(out, 0.0)
        h = jnp.dot(act, w_ref[...], preferred_element_type=jnp.float32)
        ho_ref[...] = h
        go_ref[...] = h * dis_ref[...]

    return pl.pallas_call(
        body,
        out_shape=(
            jax.ShapeDtypeStruct((N, D), jnp.float32),
            jax.ShapeDtypeStruct((N, D), jnp.float32),
        ),
    )(p, h_prev, dis, inv, b_prev, W_next)


def _tc_post(p, h_prev, dis, inv, b, batch):
    def body(p_ref, h_ref, dis_ref, inv_ref, b_ref, batch_ref, o_ref):
        acc = p_ref[0][:N, :] + p_ref[1][:N, :]
        out = dis_ref[...] * acc + inv_ref[...] * h_ref[...] + b_ref[...][None, :]
        bat = batch_ref[...]
        neg = jnp.float32(-jnp.inf)
        rows = []
        for gi in range(G):
            m = bat == gi
            rows.append(jnp.max(jnp.where(m, out, neg), axis=0, keepdims=True))
        o_ref[...] = jnp.concatenate(rows, axis=0)

    return pl.pallas_call(
        body,
        out_shape=jax.ShapeDtypeStruct((G, D), jnp.float32),
    )(p, h_prev, dis, inv, b, batch.reshape(N, 1))


def kernel(x, edge_index, batch, W1, b1, W2, b2, W3, b3):
    src = edge_index[0]
    dst = edge_index[1]
    pad_e = E_PAD - E
    src_p = jnp.concatenate([src, jnp.zeros((pad_e,), jnp.int32)])
    dst_p = jnp.concatenate([dst, jnp.full((pad_e,), N_PAD - 1, jnp.int32)])
    zeros_blk = jnp.zeros((ROWS_PER_TILE, D), jnp.float32)
    ones_blk = jnp.ones((EB, D), jnp.float32)

    degp = _deg_sc(dst_p, ones_blk, zeros_blk)
    h1, g1, dis, inv = _tc_pre(x, W1, degp)
    p1 = _spmm_sc(g1, src_p, dst_p, zeros_blk)
    h2, g2 = _tc_mid(p1, h1, dis, inv, b1, W2)
    p2 = _spmm_sc(g2, src_p, dst_p, zeros_blk)
    h3, g3 = _tc_mid(p2, h2, dis, inv, b2, W3)
    p3 = _spmm_sc(g3, src_p, dst_p, zeros_blk)
    return _tc_post(p3, h3, dis, inv, b3, batch)
